# Initial kernel scaffold; baseline (speedup 1.0000x reference)
import jax
import jax.numpy as jnp
from jax.experimental import pallas as pl


def kernel(q, v, embeddings):
    raise NotImplementedError("write your pallas kernel here")


# speedup vs baseline: 12.1396x; 12.1396x over previous
import functools

import jax
import jax.numpy as jnp
from jax import lax
from jax.experimental import pallas as pl
from jax.experimental.pallas import tpu as pltpu
from jax.experimental.pallas import tpu_sc as plsc

_NUM_CORES = 2
_NUM_SUBCORES = 16
_LANES = 16


def _rpe_call(q_len, v_len, max_len, d):
    nw = _NUM_CORES * _NUM_SUBCORES
    assert q_len % nw == 0 and d % _LANES == 0
    rpw = q_len // nw
    p = (max_len - 1) // 2
    off = (q_len - 1) - p
    span = v_len + rpw - 1
    span_pad = ((span + 3) // 4) * 4
    row_w = v_len * d
    nfire = 8
    vpr = d // _LANES

    mesh = plsc.VectorSubcoreMesh(core_axis_name="c", subcore_axis_name="s")

    @functools.partial(
        pl.kernel,
        out_type=jax.ShapeDtypeStruct((q_len, row_w), jnp.float32),
        mesh=mesh,
        compiler_params=pltpu.CompilerParams(use_tc_tiling_on_sc=False),
        scratch_types=[
            pltpu.VMEM((max_len * d,), jnp.float32),
            pltpu.VMEM((span_pad * d,), jnp.float32),
            pltpu.SemaphoreType.DMA,
            pltpu.SemaphoreType.DMA,
        ],
    )
    def rpe(emb_hbm, out_hbm, emb_v, a_v, gsem, csem):
        wid = lax.axis_index("c") * _NUM_SUBCORES + lax.axis_index("s")
        i0 = wid * rpw
        base = (q_len - 1) - i0 - (rpw - 1)
        bias = base - off

        pltpu.async_copy(emb_hbm, emb_v, gsem).wait()

        @pl.loop(0, span_pad, step=4)
        def _(t):
            for u in range(4):
                k = jnp.minimum(jnp.maximum(bias + (t + u), 0), max_len - 1)
                for h in range(vpr):
                    a_v[pl.ds((t + u) * d + h * _LANES, _LANES)] = (
                        emb_v[pl.ds(k * d + h * _LANES, _LANES)]
                    )

        @pl.loop(0, rpw // nfire)
        def _(g):
            r0 = g * nfire
            for b in range(nfire):
                r = r0 + b
                pltpu.async_copy(
                    a_v.at[pl.ds(((rpw - 1) - r) * d, row_w)],
                    out_hbm.at[i0 + r],
                    csem,
                )
            for b in range(nfire):
                r = r0 + b
                pltpu.make_async_copy(
                    a_v.at[pl.ds(((rpw - 1) - r) * d, row_w)],
                    out_hbm.at[i0 + r],
                    csem,
                ).wait()

    return rpe


def kernel(q, v, embeddings):
    q_len = int(q.shape[1])
    v_len = int(v.shape[1])
    max_len, d = int(embeddings.shape[0]), int(embeddings.shape[1])
    out = _rpe_call(q_len, v_len, max_len, d)(embeddings.reshape(max_len * d))
    return out.reshape(q_len, v_len, d)
